# Initial kernel scaffold; baseline (speedup 1.0000x reference)
import jax
import jax.numpy as jnp
from jax.experimental import pallas as pl


def kernel(x, W, b):
    raise NotImplementedError("write your pallas kernel here")


# speedup vs baseline: 5.7390x; 5.7390x over previous
import jax
import jax.numpy as jnp
from jax.experimental import pallas as pl

N_EXP = 8
BLOCK = 2048


def _gate_kernel(x_ref, w_ref, b_ref, o_ref):
    x = x_ref[...]
    w = w_ref[...]
    s = jax.lax.dot_general(
        x, w, (((1,), (1,)), ((), ())),
        preferred_element_type=jnp.float32,
    ) + b_ref[...][None, :]
    m = jnp.max(s, axis=1, keepdims=True)
    e = jnp.exp(s - m)
    denom = jnp.sum(e, axis=1, keepdims=True)
    iota = jax.lax.broadcasted_iota(jnp.int32, s.shape, 1)
    amax = jnp.min(jnp.where(s == m, iota, N_EXP), axis=1, keepdims=True)
    o_ref[...] = jnp.where(iota == amax, 1.0 / denom, 0.0)


def kernel(x, W, b):
    n_tokens, c_in = x.shape
    grid = (n_tokens // BLOCK,)
    return pl.pallas_call(
        _gate_kernel,
        grid=grid,
        in_specs=[
            pl.BlockSpec((BLOCK, c_in), lambda i: (i, 0)),
            pl.BlockSpec((N_EXP, c_in), lambda i: (0, 0)),
            pl.BlockSpec((N_EXP,), lambda i: (0,)),
        ],
        out_specs=pl.BlockSpec((BLOCK, N_EXP), lambda i: (i, 0)),
        out_shape=jax.ShapeDtypeStruct((n_tokens, N_EXP), jnp.float32),
    )(x, W, b)
